# baseline (speedup 1.0000x reference)
import functools

import jax
import jax.numpy as jnp
from jax import lax
from jax.experimental import pallas as pl
from jax.experimental.pallas import tpu as pltpu
from jax.experimental.pallas import tpu_sc as plsc

VOCAB = 100000
Q_HEADS = 16
KV_HEADS = 4
HEAD_DIM = 64
B = 32
CTX = 4096

C = 128
NCHUNK = CTX // C
D_KV = KV_HEADS * HEAD_DIM
D_Q = Q_HEADS * HEAD_DIM
NQUART = HEAD_DIM // 16


def _rope_tables():
    pos = jnp.arange(CTX + 1, dtype=jnp.float32)
    inv_freq = 1.0 / 10000.0 ** (
        jnp.arange(0, HEAD_DIM, 2, dtype=jnp.float32) / HEAD_DIM)
    freqs = pos[:, None] * inv_freq[None, :]
    emb = jnp.repeat(freqs, 2, axis=-1)
    cos = jnp.cos(emb)
    sign = jnp.where(jnp.arange(HEAD_DIM) % 2 == 0, -1.0, 1.0)
    sin_s = jnp.sin(emb) * sign[None, :]
    return cos, sin_s


def _swap_adjacent(buf, row, col_base, perm_row16, perm_col):
    return plsc.load_gather(buf, [perm_row16 + row, perm_col + col_base])


def _body(ctx_hbm, nxt_hbm, wq_hbm, wk_hbm, wv_hbm, cos_hbm, sins_hbm,
          cq_hbm, sq_hbm,
          q_hbm, k_hbm, v_hbm,
          idx_v, kbuf, vbuf, cbuf, sbuf, qidx, qbuf, qout, cqbuf, sqbuf,
          semk, semv):
    nc = 2
    wid = lax.axis_index("s") * nc + lax.axis_index("c")
    b = wid

    lane = lax.iota(jnp.int32, 16)
    perm_col = lane ^ 1
    zero16 = lane * 0

    pltpu.sync_copy(nxt_hbm.at[b], qidx)
    pltpu.async_copy(wq_hbm.at[qidx], qbuf, semk).wait()
    pltpu.sync_copy(cq_hbm, cqbuf)
    pltpu.sync_copy(sq_hbm, sqbuf)
    for j in range(D_Q // 16):
        quart = j % NQUART
        c = cqbuf[pl.ds(quart * 16, 16)]
        s = sqbuf[pl.ds(quart * 16, 16)]
        x = qbuf[0, pl.ds(j * 16, 16)]
        xs = _swap_adjacent(qbuf, 0, j * 16, zero16, perm_col)
        qout[pl.ds(j * 16, 16)] = x * c + xs * s
    pltpu.sync_copy(qout, q_hbm.at[b])

    pltpu.sync_copy(ctx_hbm.at[b], idx_v)

    def chunk(i, _):
        cpk = pltpu.async_copy(wk_hbm.at[idx_v.at[i]], kbuf, semk)
        cpv = pltpu.async_copy(wv_hbm.at[idx_v.at[i]], vbuf, semv)
        pltpu.sync_copy(cos_hbm.at[i], cbuf)
        pltpu.sync_copy(sins_hbm.at[i], sbuf)
        cpk.wait()

        def rope_t(t, carry):
            for quart in range(NQUART):
                c = cbuf[t, pl.ds(quart * 16, 16)]
                s = sbuf[t, pl.ds(quart * 16, 16)]
                for h in range(KV_HEADS):
                    off = h * HEAD_DIM + quart * 16
                    x = kbuf[t, pl.ds(off, 16)]
                    xs = _swap_adjacent(kbuf, t, off, zero16, perm_col)
                    kbuf[t, pl.ds(off, 16)] = x * c + xs * s
            return carry

        lax.fori_loop(0, C, rope_t, 0)
        cpv.wait()
        for h in range(KV_HEADS):
            pltpu.sync_copy(kbuf.at[:, pl.ds(h * HEAD_DIM, HEAD_DIM)],
                            k_hbm.at[b * KV_HEADS + h, pl.ds(i * C, C)])
            pltpu.sync_copy(vbuf.at[:, pl.ds(h * HEAD_DIM, HEAD_DIM)],
                            v_hbm.at[b * KV_HEADS + h, pl.ds(i * C, C)])
        return 0

    lax.fori_loop(0, NCHUNK, chunk, 0)


@jax.jit
def _sc_call(ctx3, nxt8, Wq, Wk, Wv, cos_k, sins_k, cq, sq):
    mesh = plsc.VectorSubcoreMesh(core_axis_name="c", subcore_axis_name="s")
    f = pl.kernel(
        _body,
        out_type=[
            jax.ShapeDtypeStruct((B, D_Q), jnp.float32),
            jax.ShapeDtypeStruct((B * KV_HEADS, CTX, HEAD_DIM), jnp.float32),
            jax.ShapeDtypeStruct((B * KV_HEADS, CTX, HEAD_DIM), jnp.float32),
        ],
        mesh=mesh,
        compiler_params=pltpu.CompilerParams(use_tc_tiling_on_sc=False,
                                             needs_layout_passes=False),
        scratch_types=[
            pltpu.VMEM((NCHUNK, C), jnp.int32),
            pltpu.VMEM((C, D_KV), jnp.float32),
            pltpu.VMEM((C, D_KV), jnp.float32),
            pltpu.VMEM((C, HEAD_DIM), jnp.float32),
            pltpu.VMEM((C, HEAD_DIM), jnp.float32),
            pltpu.VMEM((8,), jnp.int32),
            pltpu.VMEM((8, D_Q), jnp.float32),
            pltpu.VMEM((D_Q,), jnp.float32),
            pltpu.VMEM((HEAD_DIM,), jnp.float32),
            pltpu.VMEM((HEAD_DIM,), jnp.float32),
            pltpu.SemaphoreType.DMA,
            pltpu.SemaphoreType.DMA,
        ],
    )
    return f(ctx3, nxt8, Wq, Wk, Wv, cos_k, sins_k, cq, sq)


def kernel(context_tokens, next_tokens, Wq, Wk, Wv):
    cos, sin_s = _rope_tables()
    ctx3 = context_tokens.reshape(B, NCHUNK, C)
    nxt8 = jnp.broadcast_to(next_tokens[:, None], (B, 8))
    cos_k = cos[:CTX].reshape(NCHUNK, C, HEAD_DIM)
    sins_k = sin_s[:CTX].reshape(NCHUNK, C, HEAD_DIM)
    cq = cos[CTX]
    sq = sin_s[CTX]
    q, k, v = _sc_call(ctx3, nxt8, Wq, Wk, Wv, cos_k, sins_k, cq, sq)
    q = q.reshape(B, Q_HEADS, 1, HEAD_DIM)
    k = k.reshape(B, KV_HEADS, CTX, HEAD_DIM)
    v = v.reshape(B, KV_HEADS, CTX, HEAD_DIM)
    return q, k, v


# speedup vs baseline: 1.1067x; 1.1067x over previous
import jax
import jax.numpy as jnp
import numpy as np
from jax import lax
from jax.experimental import pallas as pl
from jax.experimental.pallas import tpu as pltpu
from jax.experimental.pallas import tpu_sc as plsc

VOCAB = 100000
Q_HEADS = 16
KV_HEADS = 4
HEAD_DIM = 64
B = 32
CTX = 4096

C = 64
NCHUNK = CTX // C
NBUF = 3
D_KV = KV_HEADS * HEAD_DIM
D_Q = Q_HEADS * HEAD_DIM
NQUART = HEAD_DIM // 16


def _rope_tables():
    pos = np.arange(CTX + 1, dtype=np.float64)
    inv_freq = 1.0 / 10000.0 ** (
        np.arange(0, HEAD_DIM, 2, dtype=np.float64) / HEAD_DIM)
    freqs = pos[:, None] * inv_freq[None, :]
    emb = np.repeat(freqs, 2, axis=-1)
    cos = np.cos(emb).astype(np.float32)
    sign = np.where(np.arange(HEAD_DIM) % 2 == 0, -1.0, 1.0)
    sin_s = (np.sin(emb) * sign[None, :]).astype(np.float32)
    return cos, sin_s


def _body(ctx_hbm, nxt_hbm, wq_hbm, wk_hbm, wv_hbm, cs_hbm, csq_hbm,
          q_hbm, k_hbm, v_hbm,
          idx_v, kbuf, vbuf, csbuf, qidx1, qbuf, qout, csqb,
          gsem0, gsem1, gsem2, ssem0, ssem1, ssem2):
    nc = 2
    b = lax.axis_index("s") * nc + lax.axis_index("c")
    gsem = (gsem0, gsem1, gsem2)
    ssem = (ssem0, ssem1, ssem2)
    base_h = b * KV_HEADS

    lane = lax.iota(jnp.int32, 16)
    perm_col = lane ^ 1
    zero16 = lane * 0

    def start_gather(i, nb):
        pltpu.async_copy(wk_hbm.at[idx_v.at[i]], kbuf.at[nb], gsem[nb])
        pltpu.async_copy(wv_hbm.at[idx_v.at[i]], vbuf.at[nb], gsem[nb])
        pltpu.async_copy(cs_hbm.at[i], csbuf.at[nb], gsem[nb])

    def drain_gather(nb):
        pltpu.make_async_copy(wk_hbm.at[pl.ds(0, C)], kbuf.at[nb],
                              gsem[nb]).wait()
        pltpu.make_async_copy(wv_hbm.at[pl.ds(0, C)], vbuf.at[nb],
                              gsem[nb]).wait()
        pltpu.make_async_copy(cs_hbm.at[0], csbuf.at[nb], gsem[nb]).wait()

    def start_scatter(i, nb):
        for h in range(KV_HEADS):
            pltpu.async_copy(kbuf.at[nb, :, pl.ds(h * HEAD_DIM, HEAD_DIM)],
                             k_hbm.at[base_h + h, pl.ds(i * C, C)], ssem[nb])
            pltpu.async_copy(vbuf.at[nb, :, pl.ds(h * HEAD_DIM, HEAD_DIM)],
                             v_hbm.at[base_h + h, pl.ds(i * C, C)], ssem[nb])

    def drain_scatter(nb):
        for _ in range(2 * KV_HEADS):
            pltpu.make_async_copy(
                k_hbm.at[0, pl.ds(0, C)],
                kbuf.at[nb, :, pl.ds(0, HEAD_DIM)], ssem[nb]).wait()

    def rope(nb):
        def rope_t(t, carry):
            for quart in range(NQUART):
                c = csbuf[nb, t, pl.ds(quart * 16, 16)]
                s = csbuf[nb, t, pl.ds(HEAD_DIM + quart * 16, 16)]
                for h in range(KV_HEADS):
                    off = h * HEAD_DIM + quart * 16
                    x = kbuf[nb, t, pl.ds(off, 16)]
                    xs = plsc.load_gather(
                        kbuf.at[nb], [zero16 + t, perm_col + off])
                    kbuf[nb, t, pl.ds(off, 16)] = x * c + xs * s
            return carry
        lax.fori_loop(0, C, rope_t, 0)

    def body(i, nb, prefetch, drain_prev):
        drain_gather(nb)
        rope(nb)
        start_scatter(i, nb)
        pb = (nb + 2) % NBUF
        if drain_prev:
            drain_scatter(pb)
        if prefetch:
            start_gather(i + 2, pb)

    pltpu.sync_copy(ctx_hbm.at[b], idx_v)
    start_gather(0, 0)
    start_gather(1, 1)

    pltpu.sync_copy(nxt_hbm.at[b, pl.ds(0, 1)], qidx1)
    pltpu.async_copy(wq_hbm.at[qidx1], qbuf, gsem2).wait()
    pltpu.sync_copy(csq_hbm, csqb)
    for j in range(D_Q // 16):
        quart = j % NQUART
        c = csqb[pl.ds(quart * 16, 16)]
        s = csqb[pl.ds(HEAD_DIM + quart * 16, 16)]
        x = qbuf[0, pl.ds(j * 16, 16)]
        xs = plsc.load_gather(qbuf, [zero16, perm_col + j * 16])
        qout[pl.ds(j * 16, 16)] = x * c + xs * s
    pltpu.sync_copy(qout, q_hbm.at[b])

    body(0, 0, True, False)

    def triple(g, carry):
        i = 3 * g + 1
        body(i, 1, True, True)
        body(i + 1, 2, True, True)
        body(i + 2, 0, True, True)
        return carry

    lax.fori_loop(0, (NCHUNK - 4) // 3, triple, 0)

    body(NCHUNK - 3, 1, True, True)
    body(NCHUNK - 2, 2, False, True)
    body(NCHUNK - 1, 0, False, True)
    drain_scatter(0)


@jax.jit
def _sc_call(ctx3, nxt8, Wq, Wk, Wv):
    cos, sin_s = _rope_tables()
    cs_k = np.concatenate(
        [cos[:CTX].reshape(NCHUNK, C, HEAD_DIM),
         sin_s[:CTX].reshape(NCHUNK, C, HEAD_DIM)], axis=-1)
    csq = np.concatenate([cos[CTX], sin_s[CTX]])
    mesh = plsc.VectorSubcoreMesh(core_axis_name="c", subcore_axis_name="s")
    f = pl.kernel(
        _body,
        out_type=[
            jax.ShapeDtypeStruct((B, D_Q), jnp.float32),
            jax.ShapeDtypeStruct((B * KV_HEADS, CTX, HEAD_DIM), jnp.float32),
            jax.ShapeDtypeStruct((B * KV_HEADS, CTX, HEAD_DIM), jnp.float32),
        ],
        mesh=mesh,
        compiler_params=pltpu.CompilerParams(use_tc_tiling_on_sc=False,
                                             needs_layout_passes=False),
        scratch_types=[
            pltpu.VMEM((NCHUNK, C), jnp.int32),
            pltpu.VMEM((NBUF, C, D_KV), jnp.float32),
            pltpu.VMEM((NBUF, C, D_KV), jnp.float32),
            pltpu.VMEM((NBUF, C, 2 * HEAD_DIM), jnp.float32),
            pltpu.VMEM((1,), jnp.int32),
            pltpu.VMEM((1, D_Q), jnp.float32),
            pltpu.VMEM((D_Q,), jnp.float32),
            pltpu.VMEM((2 * HEAD_DIM,), jnp.float32),
            pltpu.SemaphoreType.DMA,
            pltpu.SemaphoreType.DMA,
            pltpu.SemaphoreType.DMA,
            pltpu.SemaphoreType.DMA,
            pltpu.SemaphoreType.DMA,
            pltpu.SemaphoreType.DMA,
        ],
    )
    return f(ctx3, nxt8, Wq, Wk, Wv, jnp.asarray(cs_k), jnp.asarray(csq))


def kernel(context_tokens, next_tokens, Wq, Wk, Wv):
    ctx3 = context_tokens.reshape(B, NCHUNK, C)
    nxt8 = jnp.broadcast_to(next_tokens[:, None], (B, 8))
    q, k, v = _sc_call(ctx3, nxt8, Wq, Wk, Wv)
    q = q.reshape(B, Q_HEADS, 1, HEAD_DIM)
    k = k.reshape(B, KV_HEADS, CTX, HEAD_DIM)
    v = v.reshape(B, KV_HEADS, CTX, HEAD_DIM)
    return q, k, v
